# baseline (speedup 1.0000x reference)
import functools

import jax
import jax.numpy as jnp
from jax import lax
from jax.experimental import pallas as pl
from jax.experimental.pallas import tpu as pltpu
from jax.experimental.pallas import tpu_sc as plsc

f32 = jnp.float32
i32 = jnp.int32

NU = 60000
NI = 40000
NN = NU + NI
EH = 1_600_000
MH = 400_000
K = 100
HF = 32
NS = 16
ZC = 40
FC = 1000
BT = 2000

_mesh = plsc.VectorSubcoreMesh(core_axis_name="c", subcore_axis_name="s")
_sc_params = pltpu.CompilerParams(use_tc_tiling_on_sc=False)


def _fill_const(buf, nrows, width, value):
    vec = jnp.full((16,), value, f32)

    def body(r, carry):
        for w in range(width // 16):
            buf[r, pl.ds(w * 16, 16)] = vec
        return carry

    lax.fori_loop(0, nrows, body, 0)


def _zero_shared(acc, zb, tile, nrows, sem):
    n = nrows // ZC
    nj = n // NS + jnp.where(tile < n % NS, 1, 0)

    def issue(j):
        pltpu.async_copy(zb, acc.at[pl.ds((tile + j * NS) * ZC, ZC)], sem)

    for p in range(4):
        @pl.when(p < nj)
        def _():
            issue(p)

    def body(j, carry):
        pltpu.make_async_copy(zb, acc.at[pl.ds(0, ZC)], sem).wait()
        @pl.when(j + 4 < nj)
        def _():
            issue(j + 4)
        return carry

    lax.fori_loop(0, nj, body, 0)


def _flush_shared(acc, out_hbm, tile, nrows, obase, sem):
    n = nrows // FC
    nj = n // NS + jnp.where(tile < n % NS, 1, 0)

    def body(j, carry):
        ch = tile + j * NS
        pltpu.async_copy(acc.at[pl.ds(ch * FC, FC)],
                         out_hbm.at[pl.ds(obase + ch * FC, FC)], sem)
        return carry

    def drain(j, carry):
        pltpu.make_async_copy(acc.at[pl.ds(0, FC)],
                              out_hbm.at[pl.ds(obase, FC)], sem).wait()
        return carry

    lax.fori_loop(0, nj, body, 0)
    lax.fori_loop(0, nj, drain, 0)


def _edge_phase(cidx_hbm, x_hbm, acc, bufs, tile, cbase, nchunks):
    cidx, rows, isem, gs, ts = bufs
    nbase = nchunks // NS
    extra = nchunks % NS

    def load_idx(ch, slot, sync=False):
        if sync:
            pltpu.sync_copy(cidx_hbm.at[pl.ds(ch, 1)], cidx.at[pl.ds(slot, 1)])
        else:
            pltpu.async_copy(cidx_hbm.at[pl.ds(ch, 1)],
                             cidx.at[pl.ds(slot, 1)], isem)

    def wait_idx(ch, slot):
        pltpu.make_async_copy(cidx_hbm.at[pl.ds(ch, 1)],
                              cidx.at[pl.ds(slot, 1)], isem).wait()

    def step(j, k, do_drain, do_finish, do_load):
        q = k % 3
        q1 = (q + 1) % 3
        s_nxt = (k + 1) % 6
        s_dm3 = (k + 3) % 6
        s_fm2 = (k + 4) % 6
        ch_nxt = cbase + tile + (j + 1) * NS
        if do_drain:
            pltpu.make_async_copy(rows[q], acc.at[cidx.at[s_dm3, 0]],
                                  ts[q]).wait()
        if do_load:
            load_idx(ch_nxt, s_nxt)
        pltpu.async_copy(x_hbm.at[cidx.at[k, 1]], rows[q], gs[q])
        if do_finish:
            pltpu.make_async_copy(x_hbm.at[cidx.at[s_fm2, 1]],
                                  rows[q1], gs[q1]).wait()
            pltpu.async_copy(rows[q1], acc.at[cidx.at[s_fm2, 0]],
                             ts[q1], add=True)
        if do_load:
            wait_idx(ch_nxt, s_nxt)

    load_idx(cbase + tile, 0, sync=True)

    for k in range(6):
        step(k, k, do_drain=(k >= 3), do_finish=(k >= 2), do_load=True)
    n6 = (nbase - 6) // 6
    rem = (nbase - 6) % 6

    def group(g, carry):
        for k in range(6):
            step(6 + 6 * g + k, k, do_drain=True, do_finish=True,
                 do_load=True)
        return carry

    if rem == 0:
        lax.fori_loop(0, n6 - 1, group, 0)
        base = 6 + 6 * (n6 - 1)
        for k in range(6):
            step(base + k, k, do_drain=True, do_finish=True,
                 do_load=(k < 5))
    else:
        lax.fori_loop(0, n6, group, 0)
        base = 6 + 6 * n6
        for i in range(rem):
            step(base + i, i, do_drain=True, do_finish=True,
                 do_load=(i + 1 < rem))
    for j in (nbase - 2, nbase - 1):
        k = j % 6
        q = j % 3
        pltpu.make_async_copy(x_hbm.at[cidx.at[k, 1]], rows[q], gs[q]).wait()
        pltpu.async_copy(rows[q], acc.at[cidx.at[k, 0]], ts[q], add=True)
    for j in (nbase - 3, nbase - 2, nbase - 1):
        k = j % 6
        q = j % 3
        pltpu.make_async_copy(rows[q], acc.at[cidx.at[k, 0]], ts[q]).wait()

    if extra:
        @pl.when(tile < extra)
        def _():
            ch = cbase + NS * nbase + tile
            load_idx(ch, 0, sync=True)
            pltpu.async_copy(x_hbm.at[cidx.at[0, 1]], rows[0], gs[0]).wait()
            pltpu.sync_copy(rows[0], acc.at[cidx.at[0, 0]], add=True)


@functools.partial(
    pl.kernel,
    out_type=jax.ShapeDtypeStruct((NN, 16), f32),
    mesh=_mesh,
    compiler_params=_sc_params,
    scratch_types=[
        pltpu.VMEM((4, 2, K), i32),
        pltpu.VMEM((K, 16), f32),
        pltpu.VMEM((ZC, 16), f32),
        pltpu.VMEM_SHARED((NU, 16), f32),
        pltpu.SemaphoreType.DMA,
        pltpu.SemaphoreType.DMA,
        pltpu.SemaphoreType.DMA,
    ],
)
def _sc_deg(cidx_hbm, cnt_hbm, cidx, ones, zb, acc, isem, t0sem, t1sem):
    c = lax.axis_index("c")
    tile = lax.axis_index("s")
    _fill_const(zb, ZC, 16, 0.0)
    _fill_const(ones, K, 16, 1.0)
    ts = (t0sem, t1sem)

    nrows = NU - c * (NU - NI)
    _zero_shared(acc, zb, tile, nrows, t0sem)
    plsc.subcore_barrier()

    cbase = c * (EH // K)
    nbase = (EH // K) // NS
    extra = (EH // K) % NS

    def load_idx(ch, slot, sync=False):
        if sync:
            pltpu.sync_copy(cidx_hbm.at[pl.ds(ch, 1)], cidx.at[pl.ds(slot, 1)])
        else:
            pltpu.async_copy(cidx_hbm.at[pl.ds(ch, 1)],
                             cidx.at[pl.ds(slot, 1)], isem)

    def step(j, k, do_drain, do_load):
        q = k & 1
        s_nxt = (k + 1) & 3
        s_dm2 = (k + 2) & 3
        ch_nxt = cbase + tile + (j + 1) * NS
        if do_drain:
            pltpu.make_async_copy(ones, acc.at[cidx.at[s_dm2, 0]],
                                  ts[q]).wait()
        if do_load:
            load_idx(ch_nxt, s_nxt)
        pltpu.async_copy(ones, acc.at[cidx.at[k, 0]], ts[q], add=True)
        if do_load:
            pltpu.make_async_copy(cidx_hbm.at[pl.ds(ch_nxt, 1)],
                                  cidx.at[pl.ds(s_nxt, 1)], isem).wait()

    load_idx(cbase + tile, 0, sync=True)
    n4 = nbase // 4
    rem = nbase % 4
    for k in range(4):
        step(k, k, do_drain=(k >= 2), do_load=True)
    ng = n4 if rem > 0 else n4 - 1

    def group(g, carry):
        for k in range(4):
            step(4 * g + k, k, do_drain=True, do_load=True)
        return carry

    lax.fori_loop(1, ng, group, 0)
    if rem == 0:
        for k in range(4):
            step(4 * (n4 - 1) + k, k, do_drain=True, do_load=(k < 3))
    else:
        for i in range(rem):
            step(4 * n4 + i, i, do_drain=True, do_load=(i + 1 < rem))
    pltpu.make_async_copy(ones, acc.at[cidx.at[0, 0]], t0sem).wait()
    pltpu.make_async_copy(ones, acc.at[cidx.at[1, 0]], t1sem).wait()
    if extra:
        @pl.when(tile < extra)
        def _():
            ch = cbase + NS * nbase + tile
            load_idx(ch, 0, sync=True)
            pltpu.sync_copy(ones, acc.at[cidx.at[0, 0]], add=True)

    plsc.subcore_barrier()
    _flush_shared(acc, cnt_hbm, tile, nrows, c * NU, t0sem)


@functools.partial(
    pl.kernel,
    out_type=(jax.ShapeDtypeStruct((2 * NU, HF), f32),
              jax.ShapeDtypeStruct((2 * NI, HF), f32)),
    mesh=_mesh,
    compiler_params=_sc_params,
    scratch_types=[
        pltpu.VMEM((6, 2, K), i32),
        pltpu.VMEM((K, HF), f32),
        pltpu.VMEM((K, HF), f32),
        pltpu.VMEM((K, HF), f32),
        pltpu.VMEM_SHARED((NU, HF), f32),
        pltpu.SemaphoreType.DMA,
        pltpu.SemaphoreType.DMA,
        pltpu.SemaphoreType.DMA,
        pltpu.SemaphoreType.DMA,
        pltpu.SemaphoreType.DMA,
        pltpu.SemaphoreType.DMA,
        pltpu.SemaphoreType.DMA,
    ],
)
def _sc_layer(cidx_hbm, xu_hbm, xi_hbm, yu_hbm, yi_hbm,
              cidx, rows0, rows1, rows2, acc,
              isem, g0sem, g1sem, g2sem, t0sem, t1sem, t2sem):
    c = lax.axis_index("c")
    s = lax.axis_index("s")
    bufs = (cidx, (rows0, rows1, rows2), isem,
            (g0sem, g1sem, g2sem), (t0sem, t1sem, t2sem))
    zb = rows0.at[pl.ds(0, ZC)]

    _fill_const(rows0, ZC, HF, 0.0)
    _zero_shared(acc, zb, s, NU, t0sem)
    plsc.subcore_barrier()
    _edge_phase(cidx_hbm, xi_hbm.at[c], acc, bufs, s, 0, EH // K)
    plsc.subcore_barrier()
    _flush_shared(acc, yu_hbm, s, NU, c * NU, t0sem)
    plsc.subcore_barrier()

    _fill_const(rows0, ZC, HF, 0.0)
    _zero_shared(acc, zb, s, NI, t0sem)
    plsc.subcore_barrier()
    _edge_phase(cidx_hbm, xu_hbm.at[c], acc, bufs, s, EH // K, EH // K)
    plsc.subcore_barrier()
    _flush_shared(acc, yi_hbm, s, NI, c * NI, t0sem)


@functools.partial(
    pl.kernel,
    out_type=(jax.ShapeDtypeStruct((2 * NI, HF), f32),
              jax.ShapeDtypeStruct((2 * NI, HF), f32)),
    mesh=_mesh,
    compiler_params=_sc_params,
    scratch_types=[
        pltpu.VMEM((6, 2, K), i32),
        pltpu.VMEM((K, HF), f32),
        pltpu.VMEM((K, HF), f32),
        pltpu.VMEM((K, HF), f32),
        pltpu.VMEM_SHARED((NI, HF), f32),
        pltpu.SemaphoreType.DMA,
        pltpu.SemaphoreType.DMA,
        pltpu.SemaphoreType.DMA,
        pltpu.SemaphoreType.DMA,
        pltpu.SemaphoreType.DMA,
        pltpu.SemaphoreType.DMA,
        pltpu.SemaphoreType.DMA,
    ],
)
def _sc_h(cidx_hbm, iraw_hbm, himg_hbm, htxt_hbm,
          cidx, rows0, rows1, rows2, acc,
          isem, g0sem, g1sem, g2sem, t0sem, t1sem, t2sem):
    c = lax.axis_index("c")
    s = lax.axis_index("s")
    bufs = (cidx, (rows0, rows1, rows2), isem,
            (g0sem, g1sem, g2sem), (t0sem, t1sem, t2sem))
    zb = rows0.at[pl.ds(0, ZC)]
    for cb, out_hbm in ((0, himg_hbm), (MH // K, htxt_hbm)):
        _fill_const(rows0, ZC, HF, 0.0)
        _zero_shared(acc, zb, s, NI, t0sem)
        plsc.subcore_barrier()
        _edge_phase(cidx_hbm, iraw_hbm.at[c], acc, bufs, s, cb, MH // K)
        plsc.subcore_barrier()
        _flush_shared(acc, out_hbm, s, NI, c * NI, t0sem)
        plsc.subcore_barrier()


def _dd_from_cnt(c_ref):
    deg = c_ref[:, 0:1] * 2.0
    return jnp.where(deg > 0, lax.rsqrt(deg), 0.0)


def _tc_prep(emb, cnt, n):
    nb = n // BT

    def body(e_ref, c_ref, lo_ref, hi_ref):
        dd = _dd_from_cnt(c_ref)
        x = e_ref[...] * dd
        lo_ref[...] = x[:, :HF]
        hi_ref[...] = x[:, HF:]

    return pl.pallas_call(
        body,
        grid=(nb,),
        in_specs=[pl.BlockSpec((BT, 2 * HF), lambda i: (i, 0)),
                  pl.BlockSpec((BT, 16), lambda i: (i, 0))],
        out_specs=[pl.BlockSpec((BT, HF), lambda i: (i, 0))] * 2,
        out_shape=(jax.ShapeDtypeStruct((n, HF), f32),
                   jax.ShapeDtypeStruct((n, HF), f32)),
    )(emb, cnt)


def _tc_mid(y, cnt, n):
    nb = n // BT

    def body(y_ref, c_ref, o_ref):
        deg = c_ref[:, 0:1] * 2.0
        d2 = jnp.where(deg > 0, 1.0 / deg, 0.0)
        o_ref[...] = y_ref[...] * d2

    return pl.pallas_call(
        body,
        grid=(2, nb),
        in_specs=[pl.BlockSpec((BT, HF), lambda h, i: (h * nb + i, 0)),
                  pl.BlockSpec((BT, 16), lambda h, i: (i, 0))],
        out_specs=pl.BlockSpec((BT, HF), lambda h, i: (h * nb + i, 0)),
        out_shape=jax.ShapeDtypeStruct((2 * n, HF), f32),
    )(y, cnt)


def _tc_fin_u(emb, y1, y2, cnt):
    nb = NU // BT

    def body(e_ref, y1l, y1h, y2l, y2h, c_ref, o_ref):
        dd = _dd_from_cnt(c_ref)
        lo = e_ref[:, :HF] + dd * (y1l[...] + y2l[...])
        hi = e_ref[:, HF:] + dd * (y1h[...] + y2h[...])
        o_ref[...] = jnp.concatenate([lo, hi], axis=1) * (1.0 / 3.0)

    lo_spec = pl.BlockSpec((BT, HF), lambda i: (i, 0))
    hi_spec = pl.BlockSpec((BT, HF), lambda i: (nb + i, 0))
    return pl.pallas_call(
        body,
        grid=(nb,),
        in_specs=[pl.BlockSpec((BT, 2 * HF), lambda i: (i, 0)),
                  lo_spec, hi_spec, lo_spec, hi_spec,
                  pl.BlockSpec((BT, 16), lambda i: (i, 0))],
        out_specs=pl.BlockSpec((BT, 2 * HF), lambda i: (i, 0)),
        out_shape=jax.ShapeDtypeStruct((NU, 2 * HF), f32),
    )(emb, y1, y1, y2, y2, cnt)


def _tc_fin_i(emb, y1, y2, himg, htxt, cnt, sv):
    nb = NI // BT

    def body(e_ref, y1l, y1h, y2l, y2h, hil, hih, htl, hth, c_ref, s_ref,
             o_ref):
        dd = _dd_from_cnt(c_ref)
        si = s_ref[0, 0]
        st = s_ref[0, 1]
        lo = ((e_ref[:, :HF] + dd * (y1l[...] + y2l[...])) * (1.0 / 3.0)
              + si * hil[...] + st * htl[...])
        hi = ((e_ref[:, HF:] + dd * (y1h[...] + y2h[...])) * (1.0 / 3.0)
              + si * hih[...] + st * hth[...])
        o_ref[...] = jnp.concatenate([lo, hi], axis=1)

    lo_spec = pl.BlockSpec((BT, HF), lambda i: (i, 0))
    hi_spec = pl.BlockSpec((BT, HF), lambda i: (nb + i, 0))
    return pl.pallas_call(
        body,
        grid=(nb,),
        in_specs=[pl.BlockSpec((BT, 2 * HF), lambda i: (i, 0)),
                  lo_spec, hi_spec, lo_spec, hi_spec,
                  lo_spec, hi_spec, lo_spec, hi_spec,
                  pl.BlockSpec((BT, 16), lambda i: (i, 0)),
                  pl.BlockSpec(memory_space=pltpu.SMEM)],
        out_specs=pl.BlockSpec((BT, 2 * HF), lambda i: (i, 0)),
        out_shape=jax.ShapeDtypeStruct((NI, 2 * HF), f32),
    )(emb, y1, y1, y2, y2, himg, himg, htxt, htxt, cnt, sv)


def kernel(adj_indices, adj_values, mm_indices, mm_values, user_emb, item_emb):
    dstp = jnp.concatenate([adj_indices[0, :EH], adj_indices[0, EH:] - NU])
    srcp = jnp.concatenate([adj_indices[1, :EH] - NU, adj_indices[1, EH:]])
    cadj = jnp.stack([dstp.reshape(-1, K), srcp.reshape(-1, K)], axis=1)
    cmm = jnp.stack([mm_indices[0].reshape(-1, K),
                     mm_indices[1].reshape(-1, K)], axis=1)

    cnt = _sc_deg(cadj)
    cnt_u = cnt[:NU]
    cnt_i = cnt[NU:]

    iraw = jnp.stack([item_emb[:, :HF], item_emb[:, HF:]])
    himg, htxt = _sc_h(cmm, iraw)

    xu_lo, xu_hi = _tc_prep(user_emb, cnt_u, NU)
    xi_lo, xi_hi = _tc_prep(item_emb, cnt_i, NI)
    xu0 = jnp.stack([xu_lo, xu_hi])
    xi0 = jnp.stack([xi_lo, xi_hi])

    yu1, yi1 = _sc_layer(cadj, xu0, xi0)
    xu1 = _tc_mid(yu1, cnt_u, NU)
    xi1 = _tc_mid(yi1, cnt_i, NI)
    yu2, yi2 = _sc_layer(cadj, xu1.reshape(2, NU, HF), xi1.reshape(2, NI, HF))

    sv = jnp.stack([mm_values[0], mm_values[MH]]).reshape(1, 2)
    u_g = _tc_fin_u(user_emb, yu1, yu2, cnt_u)
    i_g = _tc_fin_i(item_emb, yi1, yi2, himg, htxt, cnt_i, sv)
    return (u_g, i_g)


# speedup vs baseline: 1.3904x; 1.3904x over previous
import functools

import jax
import jax.numpy as jnp
from jax import lax
from jax.experimental import pallas as pl
from jax.experimental.pallas import tpu as pltpu
from jax.experimental.pallas import tpu_sc as plsc

f32 = jnp.float32
i32 = jnp.int32

NU = 60000
NI = 40000
NN = NU + NI
EH = 1_600_000
MH = 400_000
K = 128
HF = 32
NS = 16
ZC = 40
FC = 1000
BT = 2000

_mesh = plsc.VectorSubcoreMesh(core_axis_name="c", subcore_axis_name="s")
_sc_params = pltpu.CompilerParams(use_tc_tiling_on_sc=False)


def _fill_const(buf, nrows, width, value):
    vec = jnp.full((16,), value, f32)

    def body(r, carry):
        for w in range(width // 16):
            buf[r, pl.ds(w * 16, 16)] = vec
        return carry

    lax.fori_loop(0, nrows, body, 0)


def _zero_shared(acc, zb, tile, nrows, sem):
    n = nrows // ZC
    nj = n // NS + jnp.where(tile < n % NS, 1, 0)

    def issue(j):
        pltpu.async_copy(zb, acc.at[pl.ds((tile + j * NS) * ZC, ZC)], sem)

    for p in range(4):
        @pl.when(p < nj)
        def _():
            issue(p)

    def body(j, carry):
        pltpu.make_async_copy(zb, acc.at[pl.ds(0, ZC)], sem).wait()
        @pl.when(j + 4 < nj)
        def _():
            issue(j + 4)
        return carry

    lax.fori_loop(0, nj, body, 0)


def _flush_shared(acc, out_hbm, tile, nrows, obase, sem):
    n = nrows // FC
    nj = n // NS + jnp.where(tile < n % NS, 1, 0)

    def body(j, carry):
        ch = tile + j * NS
        pltpu.async_copy(acc.at[pl.ds(ch * FC, FC)],
                         out_hbm.at[pl.ds(obase + ch * FC, FC)], sem)
        return carry

    def drain(j, carry):
        pltpu.make_async_copy(acc.at[pl.ds(0, FC)],
                              out_hbm.at[pl.ds(obase, FC)], sem).wait()
        return carry

    lax.fori_loop(0, nj, body, 0)
    lax.fori_loop(0, nj, drain, 0)


def _edge_phase(cidx_hbm, x_hbm, acc, bufs, tile, cbase, nchunks):
    cidx, rows, iss, gs, ts = bufs
    nbase = nchunks // NS
    extra = nchunks % NS

    def load_idx(ch, slot, sem):
        pltpu.async_copy(cidx_hbm.at[pl.ds(ch, 1)],
                         cidx.at[pl.ds(slot, 1)], sem)

    def wait_idx(ch, slot, sem):
        pltpu.make_async_copy(cidx_hbm.at[pl.ds(ch, 1)],
                              cidx.at[pl.ds(slot, 1)], sem).wait()

    def step(j, k, do_drain, do_finish, do_load, do_wait):
        q = k & 1
        qp = 1 - q
        s_pp2 = (k + 2) & 3
        s_nxt = (k + 1) & 3
        s_pm1 = (k + 3) & 3
        if do_drain:
            pltpu.make_async_copy(rows[q], acc.at[cidx.at[k, 0]],
                                  ts[q]).wait()
        if do_load:
            load_idx(cbase + tile + (j + 2) * NS, s_pp2, iss[k & 1])
        pltpu.async_copy(x_hbm.at[cidx.at[k, 1]], rows[q], gs[q])
        if do_finish:
            pltpu.make_async_copy(x_hbm.at[cidx.at[s_pm1, 1]],
                                  rows[qp], gs[qp]).wait()
            pltpu.async_copy(rows[qp], acc.at[cidx.at[s_pm1, 0]],
                             ts[qp], add=True)
        if do_wait:
            wait_idx(cbase + tile + (j + 1) * NS, s_nxt, iss[s_nxt & 1])

    def flags(j):
        return dict(do_drain=(j >= 2), do_finish=(j >= 1),
                    do_load=(j + 2 < nbase), do_wait=(j + 1 < nbase))

    pltpu.sync_copy(cidx_hbm.at[pl.ds(cbase + tile, 1)], cidx.at[pl.ds(0, 1)])
    load_idx(cbase + tile + NS, 1, iss[1])

    n4 = nbase // 4
    for k in range(4):
        step(k, k, **flags(k))

    def group(g, carry):
        for k in range(4):
            step(4 * g + k, k, do_drain=True, do_finish=True,
                 do_load=True, do_wait=True)
        return carry

    lax.fori_loop(1, n4 - 1, group, 0)
    for j in range(4 * (n4 - 1), nbase):
        step(j, j & 3, **flags(j))
    kL = (nbase - 1) & 3
    qL = (nbase - 1) & 1
    pltpu.make_async_copy(x_hbm.at[cidx.at[kL, 1]], rows[qL], gs[qL]).wait()
    pltpu.async_copy(rows[qL], acc.at[cidx.at[kL, 0]], ts[qL], add=True)
    pltpu.make_async_copy(rows[0], acc.at[cidx.at[0, 0]], ts[0]).wait()
    pltpu.make_async_copy(rows[1], acc.at[cidx.at[1, 0]], ts[1]).wait()

    if extra:
        @pl.when(tile < extra)
        def _():
            ch = cbase + NS * nbase + tile
            pltpu.sync_copy(cidx_hbm.at[pl.ds(ch, 1)], cidx.at[pl.ds(0, 1)])
            pltpu.async_copy(x_hbm.at[cidx.at[0, 1]], rows[0], gs[0]).wait()
            pltpu.sync_copy(rows[0], acc.at[cidx.at[0, 0]], add=True)


@functools.partial(
    pl.kernel,
    out_type=jax.ShapeDtypeStruct((NN, 16), f32),
    mesh=_mesh,
    compiler_params=_sc_params,
    scratch_types=[
        pltpu.VMEM((4, 2, K), i32),
        pltpu.VMEM((K, 16), f32),
        pltpu.VMEM((ZC, 16), f32),
        pltpu.VMEM_SHARED((NU, 16), f32),
        pltpu.SemaphoreType.DMA,
        pltpu.SemaphoreType.DMA,
        pltpu.SemaphoreType.DMA,
        pltpu.SemaphoreType.DMA,
    ],
)
def _sc_deg(cidx_hbm, cnt_hbm, cidx, ones, zb, acc,
            i0sem, i1sem, t0sem, t1sem):
    c = lax.axis_index("c")
    tile = lax.axis_index("s")
    _fill_const(zb, ZC, 16, 0.0)
    _fill_const(ones, K, 16, 1.0)
    iss = (i0sem, i1sem)
    ts = (t0sem, t1sem)

    nrows = NU - c * (NU - NI)
    _zero_shared(acc, zb, tile, nrows, t0sem)
    plsc.subcore_barrier()

    cbase = c * (EH // K)
    nbase = (EH // K) // NS
    extra = (EH // K) % NS

    def load_idx(ch, slot, sem):
        pltpu.async_copy(cidx_hbm.at[pl.ds(ch, 1)],
                         cidx.at[pl.ds(slot, 1)], sem)

    def step(j, k, do_drain, do_load, do_wait):
        q = k & 1
        s_nxt = (k + 1) & 3
        s_pp2 = (k + 2) & 3
        if do_drain:
            pltpu.make_async_copy(ones, acc.at[cidx.at[k, 0]],
                                  ts[q]).wait()
        if do_load:
            load_idx(cbase + tile + (j + 2) * NS, s_pp2, iss[k & 1])
        pltpu.async_copy(ones, acc.at[cidx.at[k, 0]], ts[q], add=True)
        if do_wait:
            ch_nxt = cbase + tile + (j + 1) * NS
            pltpu.make_async_copy(cidx_hbm.at[pl.ds(ch_nxt, 1)],
                                  cidx.at[pl.ds(s_nxt, 1)],
                                  iss[s_nxt & 1]).wait()

    def flags(j):
        return dict(do_drain=(j >= 2), do_load=(j + 2 < nbase),
                    do_wait=(j + 1 < nbase))

    pltpu.sync_copy(cidx_hbm.at[pl.ds(cbase + tile, 1)], cidx.at[pl.ds(0, 1)])
    load_idx(cbase + tile + NS, 1, iss[1])
    n4 = nbase // 4
    for k in range(4):
        step(k, k, **flags(k))

    def group(g, carry):
        for k in range(4):
            step(4 * g + k, k, do_drain=True, do_load=True, do_wait=True)
        return carry

    lax.fori_loop(1, n4 - 1, group, 0)
    for j in range(4 * (n4 - 1), nbase):
        step(j, j & 3, **flags(j))
    pltpu.make_async_copy(ones, acc.at[cidx.at[0, 0]], t0sem).wait()
    pltpu.make_async_copy(ones, acc.at[cidx.at[1, 0]], t1sem).wait()
    if extra:
        @pl.when(tile < extra)
        def _():
            ch = cbase + NS * nbase + tile
            pltpu.sync_copy(cidx_hbm.at[pl.ds(ch, 1)], cidx.at[pl.ds(0, 1)])
            pltpu.sync_copy(ones, acc.at[cidx.at[0, 0]], add=True)

    plsc.subcore_barrier()
    _flush_shared(acc, cnt_hbm, tile, nrows, c * NU, t0sem)


@functools.partial(
    pl.kernel,
    out_type=(jax.ShapeDtypeStruct((2 * NU, HF), f32),
              jax.ShapeDtypeStruct((2 * NI, HF), f32)),
    mesh=_mesh,
    compiler_params=_sc_params,
    scratch_types=[
        pltpu.VMEM((4, 2, K), i32),
        pltpu.VMEM((K, HF), f32),
        pltpu.VMEM((K, HF), f32),
        pltpu.VMEM((ZC, HF), f32),
        pltpu.VMEM_SHARED((NU, HF), f32),
        pltpu.SemaphoreType.DMA,
        pltpu.SemaphoreType.DMA,
        pltpu.SemaphoreType.DMA,
        pltpu.SemaphoreType.DMA,
        pltpu.SemaphoreType.DMA,
        pltpu.SemaphoreType.DMA,
    ],
)
def _sc_layer(cidx_hbm, xu_hbm, xi_hbm, yu_hbm, yi_hbm,
              cidx, rows0, rows1, zb, acc,
              i0sem, i1sem, g0sem, g1sem, t0sem, t1sem):
    c = lax.axis_index("c")
    s = lax.axis_index("s")
    bufs = (cidx, (rows0, rows1), (i0sem, i1sem),
            (g0sem, g1sem), (t0sem, t1sem))
    _fill_const(zb, ZC, HF, 0.0)

    _zero_shared(acc, zb, s, NU, t0sem)
    plsc.subcore_barrier()
    _edge_phase(cidx_hbm, xi_hbm.at[c], acc, bufs, s, 0, EH // K)
    plsc.subcore_barrier()
    _flush_shared(acc, yu_hbm, s, NU, c * NU, t0sem)
    plsc.subcore_barrier()

    _zero_shared(acc, zb, s, NI, t0sem)
    plsc.subcore_barrier()
    _edge_phase(cidx_hbm, xu_hbm.at[c], acc, bufs, s, EH // K, EH // K)
    plsc.subcore_barrier()
    _flush_shared(acc, yi_hbm, s, NI, c * NI, t0sem)


@functools.partial(
    pl.kernel,
    out_type=(jax.ShapeDtypeStruct((2 * NI, HF), f32),
              jax.ShapeDtypeStruct((2 * NI, HF), f32)),
    mesh=_mesh,
    compiler_params=_sc_params,
    scratch_types=[
        pltpu.VMEM((4, 2, K), i32),
        pltpu.VMEM((K, HF), f32),
        pltpu.VMEM((K, HF), f32),
        pltpu.VMEM((ZC, HF), f32),
        pltpu.VMEM_SHARED((NI, HF), f32),
        pltpu.SemaphoreType.DMA,
        pltpu.SemaphoreType.DMA,
        pltpu.SemaphoreType.DMA,
        pltpu.SemaphoreType.DMA,
        pltpu.SemaphoreType.DMA,
        pltpu.SemaphoreType.DMA,
    ],
)
def _sc_h(cidx_hbm, iraw_hbm, himg_hbm, htxt_hbm,
          cidx, rows0, rows1, zb, acc,
          i0sem, i1sem, g0sem, g1sem, t0sem, t1sem):
    c = lax.axis_index("c")
    s = lax.axis_index("s")
    bufs = (cidx, (rows0, rows1), (i0sem, i1sem),
            (g0sem, g1sem), (t0sem, t1sem))
    _fill_const(zb, ZC, HF, 0.0)
    for cb, out_hbm in ((0, himg_hbm), (MH // K, htxt_hbm)):
        _zero_shared(acc, zb, s, NI, t0sem)
        plsc.subcore_barrier()
        _edge_phase(cidx_hbm, iraw_hbm.at[c], acc, bufs, s, cb, MH // K)
        plsc.subcore_barrier()
        _flush_shared(acc, out_hbm, s, NI, c * NI, t0sem)
        plsc.subcore_barrier()


def _dd_from_cnt(c_ref):
    deg = c_ref[:, 0:1] * 2.0
    return jnp.where(deg > 0, lax.rsqrt(deg), 0.0)


def _tc_prep(emb, cnt, n):
    nb = n // BT

    def body(e_ref, c_ref, lo_ref, hi_ref):
        dd = _dd_from_cnt(c_ref)
        x = e_ref[...] * dd
        lo_ref[...] = x[:, :HF]
        hi_ref[...] = x[:, HF:]

    return pl.pallas_call(
        body,
        grid=(nb,),
        in_specs=[pl.BlockSpec((BT, 2 * HF), lambda i: (i, 0)),
                  pl.BlockSpec((BT, 16), lambda i: (i, 0))],
        out_specs=[pl.BlockSpec((BT, HF), lambda i: (i, 0))] * 2,
        out_shape=(jax.ShapeDtypeStruct((n, HF), f32),
                   jax.ShapeDtypeStruct((n, HF), f32)),
    )(emb, cnt)


def _tc_mid(y, cnt, n):
    nb = n // BT

    def body(y_ref, c_ref, o_ref):
        deg = c_ref[:, 0:1] * 2.0
        d2 = jnp.where(deg > 0, 1.0 / deg, 0.0)
        o_ref[...] = y_ref[...] * d2

    return pl.pallas_call(
        body,
        grid=(2, nb),
        in_specs=[pl.BlockSpec((BT, HF), lambda h, i: (h * nb + i, 0)),
                  pl.BlockSpec((BT, 16), lambda h, i: (i, 0))],
        out_specs=pl.BlockSpec((BT, HF), lambda h, i: (h * nb + i, 0)),
        out_shape=jax.ShapeDtypeStruct((2 * n, HF), f32),
    )(y, cnt)


def _tc_fin_u(emb, y1, y2, cnt):
    nb = NU // BT

    def body(e_ref, y1l, y1h, y2l, y2h, c_ref, o_ref):
        dd = _dd_from_cnt(c_ref)
        lo = e_ref[:, :HF] + dd * (y1l[...] + y2l[...])
        hi = e_ref[:, HF:] + dd * (y1h[...] + y2h[...])
        o_ref[...] = jnp.concatenate([lo, hi], axis=1) * (1.0 / 3.0)

    lo_spec = pl.BlockSpec((BT, HF), lambda i: (i, 0))
    hi_spec = pl.BlockSpec((BT, HF), lambda i: (nb + i, 0))
    return pl.pallas_call(
        body,
        grid=(nb,),
        in_specs=[pl.BlockSpec((BT, 2 * HF), lambda i: (i, 0)),
                  lo_spec, hi_spec, lo_spec, hi_spec,
                  pl.BlockSpec((BT, 16), lambda i: (i, 0))],
        out_specs=pl.BlockSpec((BT, 2 * HF), lambda i: (i, 0)),
        out_shape=jax.ShapeDtypeStruct((NU, 2 * HF), f32),
    )(emb, y1, y1, y2, y2, cnt)


def _tc_fin_i(emb, y1, y2, himg, htxt, cnt, sv):
    nb = NI // BT

    def body(e_ref, y1l, y1h, y2l, y2h, hil, hih, htl, hth, c_ref, s_ref,
             o_ref):
        dd = _dd_from_cnt(c_ref)
        si = s_ref[0, 0]
        st = s_ref[0, 1]
        lo = ((e_ref[:, :HF] + dd * (y1l[...] + y2l[...])) * (1.0 / 3.0)
              + si * hil[...] + st * htl[...])
        hi = ((e_ref[:, HF:] + dd * (y1h[...] + y2h[...])) * (1.0 / 3.0)
              + si * hih[...] + st * hth[...])
        o_ref[...] = jnp.concatenate([lo, hi], axis=1)

    lo_spec = pl.BlockSpec((BT, HF), lambda i: (i, 0))
    hi_spec = pl.BlockSpec((BT, HF), lambda i: (nb + i, 0))
    return pl.pallas_call(
        body,
        grid=(nb,),
        in_specs=[pl.BlockSpec((BT, 2 * HF), lambda i: (i, 0)),
                  lo_spec, hi_spec, lo_spec, hi_spec,
                  lo_spec, hi_spec, lo_spec, hi_spec,
                  pl.BlockSpec((BT, 16), lambda i: (i, 0)),
                  pl.BlockSpec(memory_space=pltpu.SMEM)],
        out_specs=pl.BlockSpec((BT, 2 * HF), lambda i: (i, 0)),
        out_shape=jax.ShapeDtypeStruct((NI, 2 * HF), f32),
    )(emb, y1, y1, y2, y2, himg, himg, htxt, htxt, cnt, sv)


def kernel(adj_indices, adj_values, mm_indices, mm_values, user_emb, item_emb):
    dstp = jnp.concatenate([adj_indices[0, :EH], adj_indices[0, EH:] - NU])
    srcp = jnp.concatenate([adj_indices[1, :EH] - NU, adj_indices[1, EH:]])
    cadj = jnp.stack([dstp.reshape(-1, K), srcp.reshape(-1, K)], axis=1)
    cmm = jnp.stack([mm_indices[0].reshape(-1, K),
                     mm_indices[1].reshape(-1, K)], axis=1)

    cnt = _sc_deg(cadj)
    cnt_u = cnt[:NU]
    cnt_i = cnt[NU:]

    iraw = jnp.stack([item_emb[:, :HF], item_emb[:, HF:]])
    himg, htxt = _sc_h(cmm, iraw)

    xu_lo, xu_hi = _tc_prep(user_emb, cnt_u, NU)
    xi_lo, xi_hi = _tc_prep(item_emb, cnt_i, NI)
    xu0 = jnp.stack([xu_lo, xu_hi])
    xi0 = jnp.stack([xi_lo, xi_hi])

    yu1, yi1 = _sc_layer(cadj, xu0, xi0)
    xu1 = _tc_mid(yu1, cnt_u, NU)
    xi1 = _tc_mid(yi1, cnt_i, NI)
    yu2, yi2 = _sc_layer(cadj, xu1.reshape(2, NU, HF), xi1.reshape(2, NI, HF))

    sv = jnp.stack([mm_values[0], mm_values[MH]]).reshape(1, 2)
    u_g = _tc_fin_u(user_emb, yu1, yu2, cnt_u)
    i_g = _tc_fin_i(item_emb, yi1, yi2, himg, htxt, cnt_i, sv)
    return (u_g, i_g)
